# baseline (speedup 1.0000x reference)
import functools
import math

import jax
import jax.numpy as jnp
from jax import lax
from jax.experimental import pallas as pl
from jax.experimental.pallas import tpu as pltpu
from jax.experimental.pallas import tpu_sc as plsc

D = 128
SCALE = math.sqrt(128.0)
LANES = 16
NBUF = 2
ROWS_PER_CHUNK = 2


def _sc_embed(idx3, lut, n_rows, n_cols):
    mesh = plsc.VectorSubcoreMesh(core_axis_name="c", subcore_axis_name="s")
    info = plsc.get_sparse_core_info()
    nc = info.num_cores
    nw = idx3.shape[0]
    n_chunks = idx3.shape[1]
    chunk = idx3.shape[2]
    rows_per_w = n_rows // nw

    @functools.partial(
        pl.kernel,
        mesh=mesh,
        compiler_params=pltpu.CompilerParams(use_tc_tiling_on_sc=True),
        out_type=jax.ShapeDtypeStruct((n_rows, n_cols, D), jnp.float32),
        scratch_types=[
            pltpu.VMEM((n_chunks, chunk), jnp.int32),
            pltpu.VMEM((chunk, D), jnp.float32),
            pltpu.VMEM((chunk, D), jnp.float32),
            pltpu.VMEM((chunk, D), jnp.float32),
            pltpu.VMEM((chunk, D), jnp.float32),
            pltpu.SemaphoreType.DMA,
            pltpu.SemaphoreType.DMA,
            pltpu.SemaphoreType.DMA,
            pltpu.SemaphoreType.DMA,
        ],
    )
    def k(idx_hbm, lut_hbm, out_hbm, idx_v, in0, in1, ot0, ot1, gs0, gs1, os0, os1):
        wid = lax.axis_index("s") * nc + lax.axis_index("c")
        pltpu.sync_copy(idx_hbm.at[wid], idx_v)
        row_base = wid * rows_per_w
        ins = (in0, in1)
        outs = (ot0, ot1)
        gsems = (gs0, gs1)
        osems = (os0, os1)

        pltpu.async_copy(lut_hbm.at[idx_v.at[0]], ins[0], gsems[0])
        pltpu.async_copy(lut_hbm.at[idx_v.at[1]], ins[1], gsems[1])

        def store_waits(b):
            for r in range(ROWS_PER_CHUNK):
                pltpu.make_async_copy(
                    outs[b].at[pl.ds(r * n_cols, n_cols)],
                    out_hbm.at[row_base],
                    osems[b],
                ).wait()

        def outer(g, carry):
            for b in range(NBUF):
                c = g * NBUF + b
                pltpu.make_async_copy(lut_hbm.at[idx_v.at[c]], ins[b], gsems[b]).wait()

                @pl.when(c >= NBUF)
                def _():
                    store_waits(b)

                def row_body(r, carry2):
                    for j in range(D // LANES):
                        sl = pl.ds(j * LANES, LANES)
                        outs[b][r, sl] = ins[b][r, sl] * SCALE
                    return carry2

                lax.fori_loop(0, chunk, row_body, 0)

                @pl.when(c + NBUF < n_chunks)
                def _():
                    pltpu.async_copy(
                        lut_hbm.at[idx_v.at[c + NBUF]], ins[b], gsems[b]
                    )

                for r in range(ROWS_PER_CHUNK):
                    pltpu.async_copy(
                        outs[b].at[pl.ds(r * n_cols, n_cols)],
                        out_hbm.at[row_base + c * ROWS_PER_CHUNK + r],
                        osems[b],
                    )
            return carry

        lax.fori_loop(0, n_chunks // NBUF, outer, 0)

        for b in range(NBUF):
            store_waits(b)

    return k(idx3, lut)


def kernel(x, lut):
    n_rows, n_cols = x.shape
    nw = 32
    chunk = ROWS_PER_CHUNK * n_cols
    n_chunks = (n_rows // nw) // ROWS_PER_CHUNK
    idx3 = x.reshape(nw, n_chunks, chunk).astype(jnp.int32)
    return _sc_embed(idx3, lut, n_rows, n_cols)


# speedup vs baseline: 1.7784x; 1.7784x over previous
import functools
import math

import jax
import jax.numpy as jnp
from jax import lax
from jax.experimental import pallas as pl
from jax.experimental.pallas import tpu as pltpu
from jax.experimental.pallas import tpu_sc as plsc

D = 128
SCALE = math.sqrt(128.0)
LANES = 16
NBUF = 2


def _sc_embed(idx3, lut, n_chunks, chunk, b_per_w):
    mesh = plsc.VectorSubcoreMesh(core_axis_name="c", subcore_axis_name="s")
    info = plsc.get_sparse_core_info()
    nc = info.num_cores
    B = idx3.shape[0] * idx3.shape[1] * idx3.shape[2]

    @functools.partial(
        pl.kernel,
        mesh=mesh,
        out_type=jax.ShapeDtypeStruct((B, D), jnp.float32),
        scratch_types=[
            pltpu.VMEM((n_chunks, chunk), jnp.int32),
            pltpu.VMEM((chunk, D), jnp.float32),
            pltpu.VMEM((chunk, D), jnp.float32),
            pltpu.VMEM((chunk, D), jnp.float32),
            pltpu.VMEM((chunk, D), jnp.float32),
            pltpu.SemaphoreType.DMA,
            pltpu.SemaphoreType.DMA,
            pltpu.SemaphoreType.DMA,
            pltpu.SemaphoreType.DMA,
        ],
    )
    def k(idx_hbm, lut_hbm, out_hbm, idx_v, in0, in1, ot0, ot1, gs0, gs1, os0, os1):
        wid = lax.axis_index("s") * nc + lax.axis_index("c")
        pltpu.sync_copy(idx_hbm.at[wid], idx_v)
        base = wid * b_per_w
        ins = (in0, in1)
        outs = (ot0, ot1)
        gsems = (gs0, gs1)
        osems = (os0, os1)

        pltpu.async_copy(lut_hbm.at[idx_v.at[0]], ins[0], gsems[0])
        pltpu.async_copy(lut_hbm.at[idx_v.at[1]], ins[1], gsems[1])

        def outer(g, carry):
            for b in range(NBUF):
                c = g * NBUF + b
                pltpu.make_async_copy(lut_hbm.at[idx_v.at[c]], ins[b], gsems[b]).wait()

                @pl.when(c >= NBUF)
                def _():
                    pltpu.make_async_copy(
                        outs[b], out_hbm.at[pl.ds(base, chunk)], osems[b]
                    ).wait()

                def row_body(r, carry2):
                    for j in range(D // LANES):
                        sl = pl.ds(j * LANES, LANES)
                        outs[b][r, sl] = ins[b][r, sl] * SCALE
                    return carry2

                lax.fori_loop(0, chunk, row_body, 0)

                @pl.when(c + NBUF < n_chunks)
                def _():
                    pltpu.async_copy(
                        lut_hbm.at[idx_v.at[c + NBUF]], ins[b], gsems[b]
                    )

                pltpu.async_copy(
                    outs[b], out_hbm.at[pl.ds(base + c * chunk, chunk)], osems[b]
                )
            return carry

        lax.fori_loop(0, n_chunks // NBUF, outer, 0)

        for b in range(NBUF):
            pltpu.make_async_copy(
                outs[b], out_hbm.at[pl.ds(base, chunk)], osems[b]
            ).wait()

    return k(idx3, lut)


def kernel(x, lut):
    n_rows, n_cols = x.shape
    B = n_rows * n_cols
    nw = 32
    chunk = 128
    b_per_w = B // nw
    n_chunks = b_per_w // chunk
    idx3 = x.T.reshape(nw, n_chunks, chunk).astype(jnp.int32)
    out = _sc_embed(idx3, lut, n_chunks, chunk, b_per_w)
    return out.reshape(n_cols, n_rows, D).transpose(1, 0, 2)


# speedup vs baseline: 1.7798x; 1.0008x over previous
import functools
import math

import jax
import jax.numpy as jnp
from jax import lax
from jax.experimental import pallas as pl
from jax.experimental.pallas import tpu as pltpu
from jax.experimental.pallas import tpu_sc as plsc

D = 128
SCALE = math.sqrt(128.0)
LANES = 16
NBUF = 3


def _sc_embed(idx3, lut, n_chunks, chunk, b_per_w):
    mesh = plsc.VectorSubcoreMesh(core_axis_name="c", subcore_axis_name="s")
    info = plsc.get_sparse_core_info()
    nc = info.num_cores
    B = idx3.shape[0] * idx3.shape[1] * idx3.shape[2]
    n_main = (n_chunks // NBUF) * NBUF

    @functools.partial(
        pl.kernel,
        mesh=mesh,
        out_type=jax.ShapeDtypeStruct((B, D), jnp.float32),
        scratch_types=[
            pltpu.VMEM((n_chunks, chunk), jnp.int32),
            pltpu.VMEM((chunk, D), jnp.float32),
            pltpu.VMEM((chunk, D), jnp.float32),
            pltpu.VMEM((chunk, D), jnp.float32),
            pltpu.VMEM((chunk, D), jnp.float32),
            pltpu.VMEM((chunk, D), jnp.float32),
            pltpu.VMEM((chunk, D), jnp.float32),
            pltpu.SemaphoreType.DMA,
            pltpu.SemaphoreType.DMA,
            pltpu.SemaphoreType.DMA,
            pltpu.SemaphoreType.DMA,
            pltpu.SemaphoreType.DMA,
            pltpu.SemaphoreType.DMA,
        ],
    )
    def k(idx_hbm, lut_hbm, out_hbm, idx_v,
          in0, in1, in2, ot0, ot1, ot2,
          gs0, gs1, gs2, os0, os1, os2):
        wid = lax.axis_index("s") * nc + lax.axis_index("c")
        pltpu.sync_copy(idx_hbm.at[wid], idx_v)
        base = wid * b_per_w
        ins = (in0, in1, in2)
        outs = (ot0, ot1, ot2)
        gsems = (gs0, gs1, gs2)
        osems = (os0, os1, os2)

        for b in range(NBUF):
            pltpu.async_copy(lut_hbm.at[idx_v.at[b]], ins[b], gsems[b])

        def step(c, b):
            pltpu.make_async_copy(lut_hbm.at[idx_v.at[c]], ins[b], gsems[b]).wait()

            @pl.when(c >= NBUF)
            def _():
                pltpu.make_async_copy(
                    outs[b], out_hbm.at[pl.ds(base, chunk)], osems[b]
                ).wait()

            def row_body(r, carry2):
                for j in range(D // LANES):
                    sl = pl.ds(j * LANES, LANES)
                    outs[b][r, sl] = ins[b][r, sl] * SCALE
                return carry2

            lax.fori_loop(0, chunk, row_body, 0)

            @pl.when(c + NBUF < n_chunks)
            def _():
                pltpu.async_copy(lut_hbm.at[idx_v.at[c + NBUF]], ins[b], gsems[b])

            pltpu.async_copy(
                outs[b], out_hbm.at[pl.ds(base + c * chunk, chunk)], osems[b]
            )

        def outer(g, carry):
            for b in range(NBUF):
                step(g * NBUF + b, b)
            return carry

        lax.fori_loop(0, n_main // NBUF, outer, 0)

        for c in range(n_main, n_chunks):
            step(c, c % NBUF)

        for b in range(NBUF):
            pltpu.make_async_copy(
                outs[b], out_hbm.at[pl.ds(base, chunk)], osems[b]
            ).wait()

    return k(idx3, lut)


def kernel(x, lut):
    n_rows, n_cols = x.shape
    B = n_rows * n_cols
    nw = 32
    chunk = 128
    b_per_w = B // nw
    n_chunks = b_per_w // chunk
    idx3 = x.T.reshape(nw, n_chunks, chunk).astype(jnp.int32)
    out = _sc_embed(idx3, lut, n_chunks, chunk, b_per_w)
    return out.reshape(n_cols, n_rows, D).transpose(1, 0, 2)
